# Initial kernel scaffold; baseline (speedup 1.0000x reference)
import jax
import jax.numpy as jnp
from jax.experimental import pallas as pl


def kernel(z, x):
    raise NotImplementedError("write your pallas kernel here")


# speedup vs baseline: 2.9586x; 2.9586x over previous
import functools

import jax
import jax.numpy as jnp
from jax import lax
from jax.experimental import pallas as pl
from jax.experimental.pallas import tpu as pltpu
from jax.experimental.pallas import tpu_sc as plsc

B = 64
NT = 256
NS = 1024
C = 96
H = 32
NW = 256
WS2 = 4
TOPK = 3
SEL_ROWS = TOPK * WS2
NOSEL_ROWS = (NW - TOPK) * WS2


def _roll_left(a, k, width):
    if k % width == 0:
        return a
    return pltpu.roll(a, width - (k % width), axis=1)


def _score_rank_body(z_ref, x_ref, rank_ref, resp_acc):
    b = pl.program_id(0)
    z = z_ref[0]
    x = x_ref[0]
    zm = jnp.mean(z, axis=0, keepdims=True)
    zn = zm / jnp.maximum(jnp.sqrt(jnp.sum(zm * zm, axis=1, keepdims=True)),
                          1e-12)
    s = jnp.sum(x * x, axis=1, keepdims=True)
    xn = x / jnp.maximum(jnp.sqrt(s), 1e-12)
    resp = lax.dot_general(zn, xn, (((1,), (1,)), ((), ())))
    resp_acc[pl.ds(b, 1), :] = resp

    @pl.when(b == B - 1)
    def _epilogue():
        r = resp_acc[...]
        hs = r + _roll_left(r, 1, NS)
        vs = hs + _roll_left(hs, H, NS)
        lane = lax.broadcasted_iota(jnp.int32, (B, NS), 1)
        zero = jnp.zeros((B, NS), jnp.float32)
        acc1 = zero
        for wj in range(16):
            acc1 += jnp.where((lane & 63) == wj, _roll_left(vs, wj, NS), zero)
        acc2 = zero
        for wi in range(16):
            acc2 += jnp.where((lane >> 4) == wi,
                              _roll_left(acc1, 48 * wi, NS), zero)
        wm = acc2[:, :NW] * 0.25
        lane_w = lax.broadcasted_iota(jnp.int32, (B, NW), 1)
        cnt = jnp.zeros((B, NW), jnp.int32)
        one = jnp.ones((B, NW), jnp.int32)
        izero = jnp.zeros((B, NW), jnp.int32)
        for dc in range(1, NW):
            rolled = _roll_left(wm, dc, NW)
            wrap = (lane_w + dc) >= NW
            take = (rolled > wm) | ((rolled == wm) & wrap)
            cnt += jnp.where(take, one, izero)
        rank_ref[...] = cnt


def _compute_ranks(z, x):
    return pl.pallas_call(
        _score_rank_body,
        grid=(B,),
        in_specs=[
            pl.BlockSpec((1, NT, C), lambda b: (b, 0, 0)),
            pl.BlockSpec((1, NS, C), lambda b: (b, 0, 0)),
        ],
        out_specs=pl.BlockSpec((B, NW), lambda b: (0, 0)),
        out_shape=jax.ShapeDtypeStruct((B, NW), jnp.int32),
        scratch_shapes=[pltpu.VMEM((B, NS), jnp.float32)],
    )(z, x)


def _gather_body(x_hbm, rank_hbm, sel_hbm, nosel_hbm,
                 rank_v, order_v, idx_v, rows_v, sem):
    wid = lax.axis_index("s") * 2 + lax.axis_index("c")
    for t in range(2):
        b = wid * 2 + t
        pltpu.sync_copy(rank_hbm.at[b], rank_v)
        for g in range(16):
            r_vec = rank_v[pl.ds(g * 16, 16)]
            w_vec = lax.iota(jnp.int32, 16) + (g * 16)
            plsc.store_scatter(order_v, [r_vec], w_vec)
        for g in range(64):
            r0 = lax.iota(jnp.int32, 16) + g * 16
            p = r0 >> 2
            w = plsc.load_gather(order_v, [p])
            k = r0 & 3
            tok = ((w >> 4) * 2 + (k >> 1)) * H + (w & 15) * 2 + (k & 1)
            idx_v.at[g // 8][pl.ds((g % 8) * 16, 16)] = tok + b * NS
        copies = [
            pltpu.async_copy(x_hbm.at[idx_v.at[j]],
                             rows_v.at[pl.ds(j * 128, 128)], sem)
            for j in range(8)
        ]
        for cp in copies:
            cp.wait()
        pltpu.sync_copy(rows_v.at[pl.ds(0, SEL_ROWS)],
                        sel_hbm.at[pl.ds(b * SEL_ROWS, SEL_ROWS)])
        pltpu.sync_copy(rows_v.at[pl.ds(SEL_ROWS, NOSEL_ROWS)],
                        nosel_hbm.at[pl.ds(b * NOSEL_ROWS, NOSEL_ROWS)])


@functools.cache
def _gather_windows():
    return functools.partial(
        pl.kernel,
        out_type=(
            jax.ShapeDtypeStruct((B * SEL_ROWS, C), jnp.float32),
            jax.ShapeDtypeStruct((B * NOSEL_ROWS, C), jnp.float32),
        ),
        scratch_types=[
            pltpu.VMEM((NW,), jnp.int32),
            pltpu.VMEM((NW,), jnp.int32),
            pltpu.VMEM((8, 128), jnp.int32),
            pltpu.VMEM((NS, C), jnp.float32),
            pltpu.SemaphoreType.DMA,
        ],
        mesh=plsc.VectorSubcoreMesh(core_axis_name="c", subcore_axis_name="s"),
        compiler_params=pltpu.CompilerParams(use_tc_tiling_on_sc=False,
                                             needs_layout_passes=False),
    )(_gather_body)


def kernel(z, x):
    ranks = _compute_ranks(z, x)
    x_flat = x.reshape(B * NS, C)
    sel, nosel = _gather_windows()(x_flat, ranks)
    return (sel.reshape(B * TOPK, WS2, C),
            nosel.reshape(B * (NW - TOPK), WS2, C))
